# baseline (speedup 1.0000x reference)
import functools

import jax
import jax.numpy as jnp
from jax import lax
from jax.experimental import pallas as pl
from jax.experimental.pallas import tpu as pltpu
from jax.experimental.pallas import tpu_sc as plsc

N = 10000
K = 64
F = 128
R = 16
E = N * K

NC = 2
NS = 16
NW = NC * NS
EPW = E // NW
CH = 128
CPWF = EPW // CH
TAIL = EPW - CPWF * CH
NCHK = CPWF + 1


NBUF = 4


def _gather_body(x_hbm, nb_hbm, out_hbm, idx_v, rows, sems):
    wid = lax.axis_index("s") * NC + lax.axis_index("c")
    base = wid * EPW
    pltpu.sync_copy(nb_hbm.at[wid], idx_v)
    for b in range(NBUF):
        pltpu.async_copy(x_hbm.at[idx_v.at[b]], rows[b], sems[b])

    def body(kk, carry):
        for b in range(NBUF):
            j = kk * NBUF + b
            pltpu.make_async_copy(x_hbm.at[idx_v.at[j]], rows[b], sems[b]).wait()
            pltpu.sync_copy(rows[b], out_hbm.at[pl.ds(base + j * CH, CH)])

            @pl.when(j + NBUF < CPWF)
            def _():
                pltpu.async_copy(x_hbm.at[idx_v.at[j + NBUF]], rows[b], sems[b])

        return carry

    lax.fori_loop(0, CPWF // NBUF, body, 0)
    pltpu.async_copy(x_hbm.at[idx_v.at[CPWF]], rows[0], sems[0]).wait()
    pltpu.sync_copy(
        rows[0].at[pl.ds(0, TAIL)], out_hbm.at[pl.ds(base + CPWF * CH, TAIL)]
    )


def _gather_entry(x_hbm, nb_hbm, out_hbm, idx_v, *bufs):
    rows = bufs[:NBUF]
    sems = bufs[NBUF:]
    _gather_body(x_hbm, nb_hbm, out_hbm, idx_v, rows, sems)


@functools.cache
def _sc_gather_kernel():
    return pl.kernel(
        _gather_entry,
        out_type=jax.ShapeDtypeStruct((E, F), jnp.float32),
        mesh=plsc.VectorSubcoreMesh(
            core_axis_name="c", subcore_axis_name="s", num_cores=NC, num_subcores=NS
        ),
        scratch_types=[
            pltpu.VMEM((NCHK, CH), jnp.int32),
            *[pltpu.VMEM((CH, F), jnp.float32) for _ in range(NBUF)],
            *[pltpu.SemaphoreType.DMA for _ in range(NBUF)],
        ],
    )


TN = 200
GRID = N // TN


_LOG2E = 1.4426950408889634
_LN2 = 0.6931471805599453


def _tc_body(rbf_ref, xj_ref, w1_ref, b1_ref, w2_ref, b2_ref, out_ref):
    rbf2 = rbf_ref[...].reshape(TN * K, R)
    h = jnp.dot(rbf2, w1_ref[...], preferred_element_type=jnp.float32)
    h = h + b1_ref[...]
    h = jnp.log2(1.0 + jnp.exp2(h * _LOG2E)) * _LN2
    w = jnp.dot(h, w2_ref[...], preferred_element_type=jnp.float32)
    w = w + b2_ref[...]
    prod = xj_ref[...].astype(jnp.float32) * w
    out_ref[...] = prod.reshape(TN, K, F).sum(axis=1)


def _tc_cfconv(rbf, xj, W1, b1, W2, b2):
    return pl.pallas_call(
        _tc_body,
        grid=(GRID,),
        in_specs=[
            pl.BlockSpec((TN, K, R), lambda i: (i, 0, 0)),
            pl.BlockSpec((TN * K, F), lambda i: (i, 0)),
            pl.BlockSpec((R, F), lambda i: (0, 0)),
            pl.BlockSpec((1, F), lambda i: (0, 0)),
            pl.BlockSpec((F, F), lambda i: (0, 0)),
            pl.BlockSpec((1, F), lambda i: (0, 0)),
        ],
        out_specs=pl.BlockSpec((TN, F), lambda i: (i, 0)),
        out_shape=jax.ShapeDtypeStruct((N, F), jnp.float32),
    )(rbf, xj, W1, b1, W2, b2)


def kernel(x, rbf, neighbors, W1, b1, W2, b2):
    nb = jnp.pad(
        neighbors.astype(jnp.int32).reshape(NW, EPW),
        ((0, 0), (0, NCHK * CH - EPW)),
    ).reshape(NW, NCHK, CH)
    xj = _sc_gather_kernel()(x, nb)
    return _tc_cfconv(rbf, xj, W1, b1.reshape(1, F), W2, b2.reshape(1, F))


# speedup vs baseline: 1.1137x; 1.1137x over previous
import functools

import jax
import jax.numpy as jnp
from jax import lax
from jax.experimental import pallas as pl
from jax.experimental.pallas import tpu as pltpu
from jax.experimental.pallas import tpu_sc as plsc

N = 10000
K = 64
F = 128
R = 16
E = N * K

S = 2
NP = N // S
EP = NP * K

NC = 2
NS = 16
NW = NC * NS
EPW = EP // NW
CH = 80
CPW = EPW // CH
NBUF = 5
assert CPW % NBUF == 0


def _gather_body(x_hbm, nb_hbm, out_hbm, idx_v, rows, sems):
    wid = lax.axis_index("s") * NC + lax.axis_index("c")
    base = wid * EPW
    pltpu.sync_copy(nb_hbm.at[wid], idx_v)
    for b in range(NBUF):
        pltpu.async_copy(x_hbm.at[idx_v.at[b]], rows[b], sems[b])

    def body(kk, carry):
        for b in range(NBUF):
            j = kk * NBUF + b
            pltpu.make_async_copy(x_hbm.at[idx_v.at[j]], rows[b], sems[b]).wait()
            pltpu.sync_copy(rows[b], out_hbm.at[pl.ds(base + j * CH, CH)])

            @pl.when(j + NBUF < CPW)
            def _():
                pltpu.async_copy(x_hbm.at[idx_v.at[j + NBUF]], rows[b], sems[b])

        return carry

    lax.fori_loop(0, CPW // NBUF, body, 0)


def _gather_entry(x_hbm, nb_hbm, out_hbm, idx_v, *bufs):
    rows = bufs[:NBUF]
    sems = bufs[NBUF:]
    _gather_body(x_hbm, nb_hbm, out_hbm, idx_v, rows, sems)


@functools.cache
def _sc_gather_kernel():
    return pl.kernel(
        _gather_entry,
        out_type=jax.ShapeDtypeStruct((EP, F), jnp.float32),
        mesh=plsc.VectorSubcoreMesh(
            core_axis_name="c", subcore_axis_name="s", num_cores=NC, num_subcores=NS
        ),
        scratch_types=[
            pltpu.VMEM((CPW, CH), jnp.int32),
            *[pltpu.VMEM((CH, F), jnp.float32) for _ in range(NBUF)],
            *[pltpu.SemaphoreType.DMA for _ in range(NBUF)],
        ],
    )


TN = 200
GRID = NP // TN


_LOG2E = 1.4426950408889634
_LN2 = 0.6931471805599453


def _tc_body(rbf_ref, xj_ref, w1_ref, b1_ref, w2_ref, b2_ref, out_ref):
    rbf2 = rbf_ref[...].reshape(TN * K, R)
    h = jnp.dot(rbf2, w1_ref[...], preferred_element_type=jnp.float32)
    h = h + b1_ref[...]
    h = jnp.log2(1.0 + jnp.exp2(h * _LOG2E)) * _LN2
    w = jnp.dot(h, w2_ref[...], preferred_element_type=jnp.float32)
    w = w + b2_ref[...]
    prod = xj_ref[...].astype(jnp.float32) * w
    out_ref[...] = prod.reshape(TN, K, F).sum(axis=1)


def _tc_cfconv(rbf_p, xj_p, W1, b1, W2, b2):
    return pl.pallas_call(
        _tc_body,
        grid=(GRID,),
        in_specs=[
            pl.BlockSpec((TN, K, R), lambda i: (i, 0, 0)),
            pl.BlockSpec((TN * K, F), lambda i: (i, 0)),
            pl.BlockSpec((R, F), lambda i: (0, 0)),
            pl.BlockSpec((1, F), lambda i: (0, 0)),
            pl.BlockSpec((F, F), lambda i: (0, 0)),
            pl.BlockSpec((1, F), lambda i: (0, 0)),
        ],
        out_specs=pl.BlockSpec((TN, F), lambda i: (i, 0)),
        out_shape=jax.ShapeDtypeStruct((NP, F), jnp.float32),
    )(rbf_p, xj_p, W1, b1, W2, b2)


def kernel(x, rbf, neighbors, W1, b1, W2, b2):
    nb = neighbors.astype(jnp.int32).reshape(S, NW, CPW, CH)
    b1r = b1.reshape(1, F)
    b2r = b2.reshape(1, F)
    gather = _sc_gather_kernel()
    outs = []
    for p in range(S):
        xj_p = gather(x, nb[p])
        outs.append(_tc_cfconv(rbf[p * NP:(p + 1) * NP], xj_p, W1, b1r, W2, b2r))
    return jnp.concatenate(outs, axis=0)


# speedup vs baseline: 1.1786x; 1.0583x over previous
import functools

import jax
import jax.numpy as jnp
from jax import lax
from jax.experimental import pallas as pl
from jax.experimental.pallas import tpu as pltpu
from jax.experimental.pallas import tpu_sc as plsc

N = 10000
K = 64
F = 128
R = 16
E = N * K

S = 2
NP = N // S
EP = NP * K

NC = 2
NS = 16
NW = NC * NS
EPW = EP // NW
CH = 80
CPW = EPW // CH
NBUF = 5
assert CPW % NBUF == 0


def _gather_body(x_hbm, nb_hbm, out_hbm, idx_v, rows, sems):
    wid = lax.axis_index("s") * NC + lax.axis_index("c")
    base = wid * EPW
    pltpu.sync_copy(nb_hbm.at[wid], idx_v)
    for b in range(NBUF):
        pltpu.async_copy(x_hbm.at[idx_v.at[b]], rows[b], sems[b])

    def body(kk, carry):
        for b in range(NBUF):
            j = kk * NBUF + b
            pltpu.make_async_copy(x_hbm.at[idx_v.at[j]], rows[b], sems[b]).wait()
            pltpu.sync_copy(rows[b], out_hbm.at[pl.ds(base + j * CH, CH)])

            @pl.when(j + NBUF < CPW)
            def _():
                pltpu.async_copy(x_hbm.at[idx_v.at[j + NBUF]], rows[b], sems[b])

        return carry

    lax.fori_loop(0, CPW // NBUF, body, 0)


def _gather_entry(x_hbm, nb_hbm, out_hbm, idx_v, *bufs):
    rows = bufs[:NBUF]
    sems = bufs[NBUF:]
    _gather_body(x_hbm, nb_hbm, out_hbm, idx_v, rows, sems)


@functools.cache
def _sc_gather_kernel():
    return pl.kernel(
        _gather_entry,
        out_type=jax.ShapeDtypeStruct((EP, F), jnp.float32),
        mesh=plsc.VectorSubcoreMesh(
            core_axis_name="c", subcore_axis_name="s", num_cores=NC, num_subcores=NS
        ),
        scratch_types=[
            pltpu.VMEM((CPW, CH), jnp.int32),
            *[pltpu.VMEM((CH, F), jnp.float32) for _ in range(NBUF)],
            *[pltpu.SemaphoreType.DMA for _ in range(NBUF)],
        ],
    )


TN = 200
GRID = NP // TN


_LOG2E = 1.4426950408889634
_LN2 = 0.6931471805599453


def _tc_body(rbf_ref, xj_ref, w1_ref, b1_ref, w2_ref, b2_ref, out_ref):
    rbf2 = rbf_ref[...].reshape(TN * K, R)
    h = jnp.dot(rbf2, w1_ref[...], preferred_element_type=jnp.float32)
    h = h + b1_ref[...]
    h = jnp.log2(1.0 + jnp.exp2(h * _LOG2E)) * _LN2
    w = jnp.dot(h, w2_ref[...], preferred_element_type=jnp.float32)
    w = w + b2_ref[...]
    prod = xj_ref[...].astype(jnp.float32) * w
    out_ref[...] = prod.reshape(TN, K, F).sum(axis=1)


def _tc_cfconv(p, rbf, xj_p, W1, b1, W2, b2):
    off = p * GRID
    return pl.pallas_call(
        _tc_body,
        grid=(GRID,),
        in_specs=[
            pl.BlockSpec((TN, K, R), lambda i: (i + off, 0, 0)),
            pl.BlockSpec((TN * K, F), lambda i: (i, 0)),
            pl.BlockSpec((R, F), lambda i: (0, 0)),
            pl.BlockSpec((1, F), lambda i: (0, 0)),
            pl.BlockSpec((F, F), lambda i: (0, 0)),
            pl.BlockSpec((1, F), lambda i: (0, 0)),
        ],
        out_specs=pl.BlockSpec((TN, F), lambda i: (i, 0)),
        out_shape=jax.ShapeDtypeStruct((NP, F), jnp.float32),
    )(rbf, xj_p, W1, b1, W2, b2)


def kernel(x, rbf, neighbors, W1, b1, W2, b2):
    nb = neighbors.astype(jnp.int32).reshape(S, NW, CPW, CH)
    b1r = b1.reshape(1, F)
    b2r = b2.reshape(1, F)
    gather = _sc_gather_kernel()
    outs = []
    for p in range(S):
        xj_p = gather(x, nb[p])
        outs.append(_tc_cfconv(p, rbf, xj_p, W1, b1r, W2, b2r))
    return jnp.concatenate(outs, axis=0)
